# Initial kernel scaffold; baseline (speedup 1.0000x reference)
import jax
import jax.numpy as jnp
from jax.experimental import pallas as pl


def kernel(outputs0, outputs1, outputs2, labels):
    raise NotImplementedError("write your pallas kernel here")


# speedup vs baseline: 71.5992x; 71.5992x over previous
import jax
import jax.numpy as jnp
import numpy as np
from jax.experimental import pallas as pl
from jax.experimental.pallas import tpu as pltpu

_NCLS = 80
_STRIDES = (8, 16, 32)
_LEVEL_HW = ((80, 80), (40, 40), (20, 20))
_A = sum(h * w for h, w in _LEVEL_HW)
_AP = 8448
_G = 50


def _grid_rows():
    gxs, gys, sts = [], [], []
    for (H, W), s in zip(_LEVEL_HW, _STRIDES):
        ys, xs = np.meshgrid(np.arange(H), np.arange(W), indexing="ij")
        gxs.append(xs.reshape(-1).astype(np.float32))
        gys.append(ys.reshape(-1).astype(np.float32))
        sts.append(np.full((H * W,), float(s), dtype=np.float32))
    gx = np.concatenate(gxs)
    gy = np.concatenate(gys)
    st = np.concatenate(sts)
    return np.stack([gx, gy, st], axis=0)


_AUX = _grid_rows()


def _yolox_loss_kernel(geo_ref, cls_ref, lab_ref, out_ref):
    f32 = jnp.float32
    inf = f32(jnp.inf)

    g = geo_ref[0]
    tx = g[0:1]
    ty = g[1:2]
    tw = g[2:3]
    th = g[3:4]
    obj = g[4:5]
    gx = g[5:6]
    gy = g[6:7]
    st = g[7:8]

    bx = (tx + gx) * st
    by = (ty + gy) * st
    bw = jnp.exp(tw) * st
    bh = jnp.exp(th) * st
    cxa = (gx + 0.5) * st
    cya = (gy + 0.5) * st

    lane = jax.lax.broadcasted_iota(jnp.int32, (1, _AP), 1)
    valid = lane < _A

    C = cls_ref[0]
    sig_obj = 1.0 / (1.0 + jnp.exp(-obj))
    sig_c = 1.0 / (1.0 + jnp.exp(-C))
    p = jnp.sqrt(jnp.clip(sig_c * sig_obj, 1e-8, 1.0))
    lgp = jnp.log(p)
    lg1 = jnp.log(jnp.maximum(1.0 - p, 1e-8))
    S = -jnp.sum(lg1, axis=0, keepdims=True)
    D = lg1 - lgp

    lab = lab_ref[0]
    gcx = lab[:, 0:1]
    gcy = lab[:, 1:2]
    gw = lab[:, 2:3]
    gh = lab[:, 3:4]
    gcls = lab[:, 4:5].astype(jnp.int32)
    cit = jax.lax.broadcasted_iota(jnp.int32, (_G, _NCLS), 1)
    onehot = (cit == gcls).astype(f32)

    Dg = jnp.dot(onehot, D, preferred_element_type=f32)
    CLg = jnp.dot(onehot, C, preferred_element_type=f32)

    bminx = gcx - gw * 0.5
    bmaxx = gcx + gw * 0.5
    bminy = gcy - gh * 0.5
    bmaxy = gcy + gh * 0.5
    in_box = (cxa > bminx) & (cya > bminy) & (cxa < bmaxx) & (cya < bmaxy)
    r = 2.5 * st
    in_ctr = (jnp.abs(cxa - gcx) < r) & (jnp.abs(cya - gcy) < r)
    union = (jnp.any(in_box, axis=0, keepdims=True)
             | jnp.any(in_ctr, axis=0, keepdims=True)) & valid
    inter_f = (in_box & in_ctr).astype(f32)

    px1 = bx - bw * 0.5
    px2 = bx + bw * 0.5
    py1 = by - bh * 0.5
    py2 = by + bh * 0.5
    wi = jnp.maximum(jnp.minimum(bmaxx, px2) - jnp.maximum(bminx, px1), 0.0)
    hi = jnp.maximum(jnp.minimum(bmaxy, py2) - jnp.maximum(bminy, py1), 0.0)
    inter_a = wi * hi
    iou = inter_a / (gw * gh + bw * bh - inter_a + 1e-8)

    cost = (S + Dg) - 3.0 * jnp.log(iou + 1e-8) + 100000.0 * (1.0 - inter_f)
    cost = jnp.where(union, cost, inf)
    iou_u = jnp.where(union, iou, 0.0)

    arr = iou_u
    s10 = jnp.zeros((_G, 1), f32)
    for _ in range(10):
        m = jnp.max(arr, axis=1, keepdims=True)
        s10 = s10 + jnp.maximum(m, 0.0)
        arr = jnp.where(arr >= m, -inf, arr)
    dks = jnp.maximum(s10.astype(jnp.int32), 1)

    arr2 = cost
    thresh = jnp.full((_G, 1), inf, f32)
    for i in range(10):
        m = jnp.min(arr2, axis=1, keepdims=True)
        thresh = jnp.where(dks == i + 1, m, thresh)
        arr2 = jnp.where(arr2 <= m, inf, arr2)

    matched = (cost <= thresh) & union
    msum = jnp.sum(matched.astype(f32), axis=0, keepdims=True)
    multi = msum > 1.0

    minc = jnp.min(cost, axis=0, keepdims=True)
    git = jax.lax.broadcasted_iota(jnp.int32, (_G, _AP), 0)
    gi = jnp.min(jnp.where(cost == minc, git, _G), axis=0, keepdims=True)
    onehot_gi = (git == gi).astype(f32)

    matching = jnp.where(multi, onehot_gi, matched.astype(f32))
    fgf = (msum > 0.0).astype(f32)

    miou = jnp.sum(matching * iou, axis=0, keepdims=True)
    tbx = jnp.sum(matching * gcx, axis=0, keepdims=True)
    tby = jnp.sum(matching * gcy, axis=0, keepdims=True)
    tbw = jnp.sum(matching * gw, axis=0, keepdims=True)
    tbh = jnp.sum(matching * gh, axis=0, keepdims=True)
    xg = jnp.sum(matching * CLg, axis=0, keepdims=True)

    wi2 = jnp.maximum(jnp.minimum(tbx + tbw * 0.5, px2)
                      - jnp.maximum(tbx - tbw * 0.5, px1), 0.0)
    hi2 = jnp.maximum(jnp.minimum(tby + tbh * 0.5, py2)
                      - jnp.maximum(tby - tbh * 0.5, py1), 0.0)
    inter2 = wi2 * hi2
    iou2 = inter2 / (bw * bh + tbw * tbh - inter2 + 1e-8)
    l_iou = jnp.sum(fgf * (1.0 - iou2 * iou2), axis=1, keepdims=True)

    bce_obj = jnp.maximum(obj, 0.0) - obj * fgf + jnp.log1p(jnp.exp(-jnp.abs(obj)))
    l_obj = jnp.sum(jnp.where(valid, bce_obj, 0.0), axis=1, keepdims=True)

    bsum = jnp.sum(jnp.maximum(C, 0.0) + jnp.log1p(jnp.exp(-jnp.abs(C))),
                   axis=0, keepdims=True)
    l_cls = jnp.sum(fgf * (bsum - miou * xg), axis=1, keepdims=True)
    nfg = jnp.sum(fgf, axis=1, keepdims=True)

    li = jax.lax.broadcasted_iota(jnp.int32, (1, 128), 1)
    vec = jnp.where(li == 0, l_iou, 0.0)
    vec = jnp.where(li == 1, l_obj, vec)
    vec = jnp.where(li == 2, l_cls, vec)
    vec = jnp.where(li == 3, nfg, vec)
    out_ref[0] = vec


def kernel(outputs0, outputs1, outputs2, labels):
    bs = outputs0.shape[0]
    ch = outputs0.shape[1]
    X = jnp.concatenate(
        [outputs0.reshape(bs, ch, -1),
         outputs1.reshape(bs, ch, -1),
         outputs2.reshape(bs, ch, -1)], axis=2)
    aux = jnp.asarray(_AUX)
    geo = jnp.concatenate(
        [X[:, :5, :], jnp.broadcast_to(aux[None], (bs, 3, _A))], axis=1)
    geo = jnp.pad(geo, ((0, 0), (0, 0), (0, _AP - _A)))
    clsp = jnp.pad(X[:, 5:, :], ((0, 0), (0, 0), (0, _AP - _A)))

    out = pl.pallas_call(
        _yolox_loss_kernel,
        grid=(bs,),
        in_specs=[
            pl.BlockSpec((1, 8, _AP), lambda b: (b, 0, 0)),
            pl.BlockSpec((1, _NCLS, _AP), lambda b: (b, 0, 0)),
            pl.BlockSpec((1, _G, 5), lambda b: (b, 0, 0)),
        ],
        out_specs=pl.BlockSpec((1, 1, 128), lambda b: (b, 0, 0)),
        out_shape=jax.ShapeDtypeStruct((bs, 1, 128), jnp.float32),
        compiler_params=pltpu.CompilerParams(
            dimension_semantics=("parallel",)),
    )(geo, clsp, labels)

    s = out.reshape(bs, 128).sum(axis=0)
    return (5.0 * s[0] + s[1] + s[2]) / jnp.maximum(s[3], 1.0)


# speedup vs baseline: 74.8832x; 1.0459x over previous
import jax
import jax.numpy as jnp
import numpy as np
from jax.experimental import pallas as pl
from jax.experimental.pallas import tpu as pltpu

_NCLS = 80
_STRIDES = (8, 16, 32)
_LEVEL_HW = ((80, 80), (40, 40), (20, 20))
_A = sum(h * w for h, w in _LEVEL_HW)
_AP = 8448
_G = 50


def _grid_rows():
    gxs, gys, sts = [], [], []
    for (H, W), s in zip(_LEVEL_HW, _STRIDES):
        ys, xs = np.meshgrid(np.arange(H), np.arange(W), indexing="ij")
        gxs.append(xs.reshape(-1).astype(np.float32))
        gys.append(ys.reshape(-1).astype(np.float32))
        sts.append(np.full((H * W,), float(s), dtype=np.float32))
    gx = np.concatenate(gxs)
    gy = np.concatenate(gys)
    st = np.concatenate(sts)
    return np.stack([gx, gy, st], axis=0)


_AUX = _grid_rows()


def _yolox_loss_kernel(geo_ref, cls_ref, lab_ref, out_ref):
    f32 = jnp.float32
    inf = f32(jnp.inf)

    g = geo_ref[0]
    tx = g[0:1]
    ty = g[1:2]
    tw = g[2:3]
    th = g[3:4]
    obj = g[4:5]
    gx = g[5:6]
    gy = g[6:7]
    st = g[7:8]

    bx = (tx + gx) * st
    by = (ty + gy) * st
    bw = jnp.exp(tw) * st
    bh = jnp.exp(th) * st
    cxa = (gx + 0.5) * st
    cya = (gy + 0.5) * st

    lane = jax.lax.broadcasted_iota(jnp.int32, (1, _AP), 1)
    valid = lane < _A

    C = cls_ref[0]
    sig_obj = 1.0 / (1.0 + jnp.exp(-obj))
    e1 = jnp.exp(-jnp.abs(C))
    rc = 1.0 / (1.0 + e1)
    sig_c = jnp.where(C >= 0, rc, e1 * rc)
    bsum = jnp.sum(jnp.maximum(C, 0.0) + jnp.log1p(e1), axis=0, keepdims=True)
    p = jnp.sqrt(jnp.clip(sig_c * sig_obj, 1e-8, 1.0))
    lgp = jnp.log(p)
    lg1 = jnp.log(jnp.maximum(1.0 - p, 1e-8))
    S = -jnp.sum(lg1, axis=0, keepdims=True)
    D = lg1 - lgp

    lab = lab_ref[0]
    gcx = lab[:, 0:1]
    gcy = lab[:, 1:2]
    gw = lab[:, 2:3]
    gh = lab[:, 3:4]
    gcls = lab[:, 4:5].astype(jnp.int32)
    cit = jax.lax.broadcasted_iota(jnp.int32, (_G, _NCLS), 1)
    onehot = (cit == gcls).astype(f32)

    Dg = jnp.dot(onehot, D, preferred_element_type=f32)
    CLg = jnp.dot(onehot, C, preferred_element_type=f32)

    bminx = gcx - gw * 0.5
    bmaxx = gcx + gw * 0.5
    bminy = gcy - gh * 0.5
    bmaxy = gcy + gh * 0.5
    in_box = (cxa > bminx) & (cya > bminy) & (cxa < bmaxx) & (cya < bmaxy)
    r = 2.5 * st
    in_ctr = (jnp.abs(cxa - gcx) < r) & (jnp.abs(cya - gcy) < r)
    union = (jnp.any(in_box, axis=0, keepdims=True)
             | jnp.any(in_ctr, axis=0, keepdims=True)) & valid
    inter_f = (in_box & in_ctr).astype(f32)

    px1 = bx - bw * 0.5
    px2 = bx + bw * 0.5
    py1 = by - bh * 0.5
    py2 = by + bh * 0.5
    wi = jnp.maximum(jnp.minimum(bmaxx, px2) - jnp.maximum(bminx, px1), 0.0)
    hi = jnp.maximum(jnp.minimum(bmaxy, py2) - jnp.maximum(bminy, py1), 0.0)
    inter_a = wi * hi
    iou = inter_a / (gw * gh + bw * bh - inter_a + 1e-8)

    cost = (S + Dg) - 3.0 * jnp.log(iou + 1e-8) + 100000.0 * (1.0 - inter_f)
    cost = jnp.where(union, cost, inf)
    iou_u = jnp.where(union, iou, 0.0)

    nchunk = _AP // 128

    tmax = [jnp.full((_G, 128), -inf, f32) for _ in range(10)]
    for c in range(nchunk):
        x = iou_u[:, c * 128:(c + 1) * 128]
        for j in range(10):
            hi = jnp.maximum(tmax[j], x)
            x = jnp.minimum(tmax[j], x)
            tmax[j] = hi
    cand_iou = jnp.concatenate(tmax, axis=1)
    s10 = jnp.zeros((_G, 1), f32)
    for _ in range(10):
        m = jnp.max(cand_iou, axis=1, keepdims=True)
        s10 = s10 + jnp.maximum(m, 0.0)
        cand_iou = jnp.where(cand_iou >= m, -inf, cand_iou)
    dks = jnp.maximum(s10.astype(jnp.int32), 1)

    tmin = [jnp.full((_G, 128), inf, f32) for _ in range(10)]
    for c in range(nchunk):
        x = cost[:, c * 128:(c + 1) * 128]
        for j in range(10):
            lo = jnp.minimum(tmin[j], x)
            x = jnp.maximum(tmin[j], x)
            tmin[j] = lo
    cand_cost = jnp.concatenate(tmin, axis=1)
    thresh = jnp.full((_G, 1), inf, f32)
    for i in range(10):
        m = jnp.min(cand_cost, axis=1, keepdims=True)
        thresh = jnp.where(dks == i + 1, m, thresh)
        cand_cost = jnp.where(cand_cost <= m, inf, cand_cost)

    matched = (cost <= thresh) & union
    msum = jnp.sum(matched.astype(f32), axis=0, keepdims=True)
    multi = msum > 1.0

    minc = jnp.min(cost, axis=0, keepdims=True)
    git = jax.lax.broadcasted_iota(jnp.int32, (_G, _AP), 0)
    gi = jnp.min(jnp.where(cost == minc, git, _G), axis=0, keepdims=True)
    onehot_gi = (git == gi).astype(f32)

    matching = jnp.where(multi, onehot_gi, matched.astype(f32))
    fgf = (msum > 0.0).astype(f32)

    miou = jnp.sum(matching * iou, axis=0, keepdims=True)
    tbx = jnp.sum(matching * gcx, axis=0, keepdims=True)
    tby = jnp.sum(matching * gcy, axis=0, keepdims=True)
    tbw = jnp.sum(matching * gw, axis=0, keepdims=True)
    tbh = jnp.sum(matching * gh, axis=0, keepdims=True)
    xg = jnp.sum(matching * CLg, axis=0, keepdims=True)

    wi2 = jnp.maximum(jnp.minimum(tbx + tbw * 0.5, px2)
                      - jnp.maximum(tbx - tbw * 0.5, px1), 0.0)
    hi2 = jnp.maximum(jnp.minimum(tby + tbh * 0.5, py2)
                      - jnp.maximum(tby - tbh * 0.5, py1), 0.0)
    inter2 = wi2 * hi2
    iou2 = inter2 / (bw * bh + tbw * tbh - inter2 + 1e-8)
    l_iou = jnp.sum(fgf * (1.0 - iou2 * iou2), axis=1, keepdims=True)

    bce_obj = jnp.maximum(obj, 0.0) - obj * fgf + jnp.log1p(jnp.exp(-jnp.abs(obj)))
    l_obj = jnp.sum(jnp.where(valid, bce_obj, 0.0), axis=1, keepdims=True)

    l_cls = jnp.sum(fgf * (bsum - miou * xg), axis=1, keepdims=True)
    nfg = jnp.sum(fgf, axis=1, keepdims=True)

    li = jax.lax.broadcasted_iota(jnp.int32, (1, 128), 1)
    vec = jnp.where(li == 0, l_iou, 0.0)
    vec = jnp.where(li == 1, l_obj, vec)
    vec = jnp.where(li == 2, l_cls, vec)
    vec = jnp.where(li == 3, nfg, vec)
    out_ref[0] = vec


def kernel(outputs0, outputs1, outputs2, labels):
    bs = outputs0.shape[0]
    ch = outputs0.shape[1]
    X = jnp.concatenate(
        [outputs0.reshape(bs, ch, -1),
         outputs1.reshape(bs, ch, -1),
         outputs2.reshape(bs, ch, -1)], axis=2)
    aux = jnp.asarray(_AUX)
    geo = jnp.concatenate(
        [X[:, :5, :], jnp.broadcast_to(aux[None], (bs, 3, _A))], axis=1)
    geo = jnp.pad(geo, ((0, 0), (0, 0), (0, _AP - _A)))
    clsp = jnp.pad(X[:, 5:, :], ((0, 0), (0, 0), (0, _AP - _A)))

    out = pl.pallas_call(
        _yolox_loss_kernel,
        grid=(bs,),
        in_specs=[
            pl.BlockSpec((1, 8, _AP), lambda b: (b, 0, 0)),
            pl.BlockSpec((1, _NCLS, _AP), lambda b: (b, 0, 0)),
            pl.BlockSpec((1, _G, 5), lambda b: (b, 0, 0)),
        ],
        out_specs=pl.BlockSpec((1, 1, 128), lambda b: (b, 0, 0)),
        out_shape=jax.ShapeDtypeStruct((bs, 1, 128), jnp.float32),
        compiler_params=pltpu.CompilerParams(
            dimension_semantics=("parallel",)),
    )(geo, clsp, labels)

    s = out.reshape(bs, 128).sum(axis=0)
    return (5.0 * s[0] + s[1] + s[2]) / jnp.maximum(s[3], 1.0)


# speedup vs baseline: 105.1147x; 1.4037x over previous
import jax
import jax.numpy as jnp
from jax.experimental import pallas as pl
from jax.experimental.pallas import tpu as pltpu

_NCLS = 80
_STRIDES = (8, 16, 32)
_LEVEL_HW = ((80, 80), (40, 40), (20, 20))
_A = sum(h * w for h, w in _LEVEL_HW)
_AP = 8448
_G = 50
_HALF_LOG_CLIP = -9.210340371976184


def _grid_const_rows():
    gxs, gys, sts = [], [], []
    for (H, W), s in zip(_LEVEL_HW, _STRIDES):
        i = jax.lax.broadcasted_iota(jnp.int32, (1, H * W), 1).astype(jnp.float32)
        y = jnp.floor(i / W)
        gxs.append(i - y * W)
        gys.append(y)
        sts.append(jnp.full((1, H * W), float(s), jnp.float32))
    pad = jnp.zeros((1, _AP - _A), jnp.float32)
    gx = jnp.concatenate(gxs + [pad], axis=1)
    gy = jnp.concatenate(gys + [pad], axis=1)
    st = jnp.concatenate(sts + [pad], axis=1)
    return gx, gy, st


def _yolox_loss_kernel(o0_ref, o1_ref, o2_ref, lab_ref, labt_ref, out_ref):
    f32 = jnp.float32
    inf = f32(jnp.inf)

    zpad = jnp.zeros((5 + _NCLS, _AP - _A), f32)
    X = jnp.concatenate([o0_ref[0], o1_ref[0], o2_ref[0], zpad], axis=1)
    gx, gy, st = _grid_const_rows()

    tx = X[0:1]
    ty = X[1:2]
    tw = X[2:3]
    th = X[3:4]
    obj = X[4:5]
    C = X[5:]

    bx = (tx + gx) * st
    by = (ty + gy) * st
    bw = jnp.exp(tw) * st
    bh = jnp.exp(th) * st
    cxa = (gx + 0.5) * st
    cya = (gy + 0.5) * st

    lane = jax.lax.broadcasted_iota(jnp.int32, (1, _AP), 1)
    valid = lane < _A

    rC = jnp.maximum(C, 0.0)
    e1 = jnp.exp(-jnp.abs(C))
    l1 = jnp.log1p(e1)
    bsum_el = rC + l1
    sp_negC = bsum_el - C
    ro = jnp.maximum(obj, 0.0)
    l1o = jnp.log1p(jnp.exp(-jnp.abs(obj)))
    sp_nego = ro - obj + l1o

    lgp = jnp.maximum(-0.5 * (sp_negC + sp_nego), _HALF_LOG_CLIP)
    p = jnp.exp(lgp)
    lg1 = jnp.log(jnp.maximum(1.0 - p, 1e-8))
    S = -jnp.sum(lg1, axis=0, keepdims=True)
    D = lg1 - lgp
    bsum = jnp.sum(bsum_el, axis=0, keepdims=True)

    lab = lab_ref[0]
    gcx = lab[:, 0:1]
    gcy = lab[:, 1:2]
    gw = lab[:, 2:3]
    gh = lab[:, 3:4]
    gcls = lab[:, 4:5].astype(jnp.int32)
    cit = jax.lax.broadcasted_iota(jnp.int32, (_G, _NCLS), 1)
    onehot = (cit == gcls).astype(f32)

    Dg = jnp.dot(onehot, D, preferred_element_type=f32)
    CLg = jnp.dot(onehot, C, preferred_element_type=f32)

    bminx = gcx - gw * 0.5
    bmaxx = gcx + gw * 0.5
    bminy = gcy - gh * 0.5
    bmaxy = gcy + gh * 0.5
    in_box = (cxa > bminx) & (cya > bminy) & (cxa < bmaxx) & (cya < bmaxy)
    r = 2.5 * st
    in_ctr = (jnp.abs(cxa - gcx) < r) & (jnp.abs(cya - gcy) < r)
    union = (jnp.any(in_box, axis=0, keepdims=True)
             | jnp.any(in_ctr, axis=0, keepdims=True)) & valid
    inter_f = (in_box & in_ctr).astype(f32)

    px1 = bx - bw * 0.5
    px2 = bx + bw * 0.5
    py1 = by - bh * 0.5
    py2 = by + bh * 0.5
    wi = jnp.maximum(jnp.minimum(bmaxx, px2) - jnp.maximum(bminx, px1), 0.0)
    hi = jnp.maximum(jnp.minimum(bmaxy, py2) - jnp.maximum(bminy, py1), 0.0)
    inter_a = wi * hi
    iou = inter_a / (gw * gh + bw * bh - inter_a + 1e-8)

    cost = (S + Dg) - 3.0 * jnp.log(iou + 1e-8) + 100000.0 * (1.0 - inter_f)
    cost = jnp.where(union, cost, inf)
    iou_u = jnp.where(union, iou, 0.0)

    nchunk = _AP // 128

    tmax = [jnp.full((_G, 128), -inf, f32) for _ in range(10)]
    for c in range(nchunk):
        x = iou_u[:, c * 128:(c + 1) * 128]
        for j in range(10):
            hi2 = jnp.maximum(tmax[j], x)
            x = jnp.minimum(tmax[j], x)
            tmax[j] = hi2
    cand_iou = jnp.concatenate(tmax, axis=1)
    s10 = jnp.zeros((_G, 1), f32)
    for _ in range(10):
        m = jnp.max(cand_iou, axis=1, keepdims=True)
        s10 = s10 + jnp.maximum(m, 0.0)
        cand_iou = jnp.where(cand_iou >= m, -inf, cand_iou)
    dks = jnp.maximum(s10.astype(jnp.int32), 1)

    tmin = [jnp.full((_G, 128), inf, f32) for _ in range(10)]
    for c in range(nchunk):
        x = cost[:, c * 128:(c + 1) * 128]
        for j in range(10):
            lo = jnp.minimum(tmin[j], x)
            x = jnp.maximum(tmin[j], x)
            tmin[j] = lo
    cand_cost = jnp.concatenate(tmin, axis=1)
    thresh = jnp.full((_G, 1), inf, f32)
    for i in range(10):
        m = jnp.min(cand_cost, axis=1, keepdims=True)
        thresh = jnp.where(dks == i + 1, m, thresh)
        cand_cost = jnp.where(cand_cost <= m, inf, cand_cost)

    matched = (cost <= thresh) & union
    msum = jnp.sum(matched.astype(f32), axis=0, keepdims=True)
    multi = msum > 1.0

    minc = jnp.min(cost, axis=0, keepdims=True)
    onehot_gi = (cost == minc).astype(f32)

    matching = jnp.where(multi, onehot_gi, matched.astype(f32))
    fgf = (msum > 0.0).astype(f32)

    miou = jnp.sum(matching * iou, axis=0, keepdims=True)
    xg = jnp.sum(matching * CLg, axis=0, keepdims=True)
    boxT = labt_ref[0][0:4]
    tb = jnp.dot(boxT, matching, preferred_element_type=f32)
    tbx = tb[0:1]
    tby = tb[1:2]
    tbw = tb[2:3]
    tbh = tb[3:4]

    wi3 = jnp.maximum(jnp.minimum(tbx + tbw * 0.5, px2)
                      - jnp.maximum(tbx - tbw * 0.5, px1), 0.0)
    hi3 = jnp.maximum(jnp.minimum(tby + tbh * 0.5, py2)
                      - jnp.maximum(tby - tbh * 0.5, py1), 0.0)
    inter2 = wi3 * hi3
    iou2 = inter2 / (bw * bh + tbw * tbh - inter2 + 1e-8)
    l_iou = jnp.sum(fgf * (1.0 - iou2 * iou2), axis=1, keepdims=True)

    bce_obj = ro - obj * fgf + l1o
    l_obj = jnp.sum(jnp.where(valid, bce_obj, 0.0), axis=1, keepdims=True)

    l_cls = jnp.sum(fgf * (bsum - miou * xg), axis=1, keepdims=True)
    nfg = jnp.sum(fgf, axis=1, keepdims=True)

    li = jax.lax.broadcasted_iota(jnp.int32, (1, 128), 1)
    vec = jnp.where(li == 0, l_iou, 0.0)
    vec = jnp.where(li == 1, l_obj, vec)
    vec = jnp.where(li == 2, l_cls, vec)
    vec = jnp.where(li == 3, nfg, vec)
    out_ref[0] = vec


def kernel(outputs0, outputs1, outputs2, labels):
    bs = outputs0.shape[0]
    ch = outputs0.shape[1]
    labt = jnp.transpose(labels, (0, 2, 1))

    out = pl.pallas_call(
        _yolox_loss_kernel,
        grid=(bs,),
        in_specs=[
            pl.BlockSpec((1, ch, 6400), lambda b: (b, 0, 0)),
            pl.BlockSpec((1, ch, 1600), lambda b: (b, 0, 0)),
            pl.BlockSpec((1, ch, 400), lambda b: (b, 0, 0)),
            pl.BlockSpec((1, _G, 5), lambda b: (b, 0, 0)),
            pl.BlockSpec((1, 5, _G), lambda b: (b, 0, 0)),
        ],
        out_specs=pl.BlockSpec((1, 1, 128), lambda b: (b, 0, 0)),
        out_shape=jax.ShapeDtypeStruct((bs, 1, 128), jnp.float32),
        compiler_params=pltpu.CompilerParams(
            dimension_semantics=("parallel",)),
    )(outputs0.reshape(bs, ch, -1),
      outputs1.reshape(bs, ch, -1),
      outputs2.reshape(bs, ch, -1),
      labels, labt)

    s = out.reshape(bs, 128).sum(axis=0)
    return (5.0 * s[0] + s[1] + s[2]) / jnp.maximum(s[3], 1.0)
